# Initial kernel scaffold; baseline (speedup 1.0000x reference)
import jax
import jax.numpy as jnp
from jax.experimental import pallas as pl


def kernel(x, edge_index, W1, a_src1, a_dst1, b1, W2, a_src2, a_dst2, b2):
    raise NotImplementedError("write your pallas kernel here")


# speedup vs baseline: 11.5079x; 11.5079x over previous
import functools

import jax
import jax.numpy as jnp
from jax import lax
from jax.experimental import pallas as pl
from jax.experimental.pallas import tpu as pltpu
from jax.experimental.pallas import tpu_sc as plsc

N = 10000
E = 320000
EE = E + N
NW = 32
CHUNK = 128
NCH = 81
EW = NCH * CHUNK
EP = NW * EW
NP = 10240
NSLICE = NP // 16
L = 16
CB = 64

_mesh = functools.partial(
    plsc.VectorSubcoreMesh, core_axis_name="c", subcore_axis_name="s",
    num_cores=2, num_subcores=16)


def _tc_layer1(x, W1, a_src1, a_dst1):
    def body(x_ref, w_ref, asr_ref, adr_ref, *outs):
        h = jnp.dot(x_ref[...], w_ref[...], preferred_element_type=jnp.float32)
        for i in range(256 // CB):
            outs[i][...] = h[:, i * CB:(i + 1) * CB]
        outs[-2][...] = jnp.sum(h * asr_ref[...][None, :], axis=1)
        outs[-1][...] = jnp.sum(h * adr_ref[...][None, :], axis=1)

    return pl.pallas_call(
        body,
        out_shape=(
            tuple(jax.ShapeDtypeStruct((N, CB), jnp.float32)
                  for _ in range(256 // CB))
            + (jax.ShapeDtypeStruct((N,), jnp.float32),
               jax.ShapeDtypeStruct((N,), jnp.float32))
        ),
    )(x, W1, a_src1, a_dst1)


def _tc_layer2(p1, b1, W2, a_src2, a_dst2):
    R = 2048

    def body(*refs):
        p_refs = refs[:len(p1)]
        b_ref, w_ref, asr_ref, adr_ref = refs[len(p1):len(p1) + 4]
        h2_refs = refs[len(p1) + 4:-2]
        as_ref, ad_ref = refs[-2:]
        o = jnp.concatenate(
            [jnp.maximum(
                p[0] + p[1] + b_ref[...][None, i * CB:(i + 1) * CB], 0.0)
             for i, p in enumerate(pr[...] for pr in p_refs)], axis=1)
        h2 = jnp.dot(o, w_ref[...], preferred_element_type=jnp.float32)
        for i in range(128 // CB):
            h2_refs[i][...] = h2[:, i * CB:(i + 1) * CB]
        pid = pl.program_id(0)
        as_ref[pl.ds(pid * R, R)] = jnp.sum(h2 * asr_ref[...][None, :], axis=1)
        ad_ref[pl.ds(pid * R, R)] = jnp.sum(h2 * adr_ref[...][None, :], axis=1)

    nslab1 = len(p1)
    return pl.pallas_call(
        body,
        grid=(NP // R,),
        in_specs=(
            [pl.BlockSpec((2, R, CB), lambda i: (0, i, 0))] * nslab1
            + [pl.BlockSpec((256,), lambda i: (0,)),
               pl.BlockSpec((256, 128), lambda i: (0, 0)),
               pl.BlockSpec((128,), lambda i: (0,)),
               pl.BlockSpec((128,), lambda i: (0,))]
        ),
        out_specs=(
            [pl.BlockSpec((R, CB), lambda i: (i, 0))] * (128 // CB)
            + [pl.BlockSpec((NP,), lambda i: (0,)),
               pl.BlockSpec((NP,), lambda i: (0,))]
        ),
        out_shape=(
            tuple(jax.ShapeDtypeStruct((NP, CB), jnp.float32)
                  for _ in range(128 // CB))
            + (jax.ShapeDtypeStruct((NP,), jnp.float32),
               jax.ShapeDtypeStruct((NP,), jnp.float32))
        ),
    )(*p1, b1, W2, a_src2, a_dst2)


def _tc_final(p2, b2):
    def body(*refs):
        p_refs = refs[:len(p2)]
        b_ref = refs[len(p2)]
        out_ref = refs[-1]
        out_ref[...] = jnp.concatenate(
            [jnp.maximum(
                p[0, :N, :] + p[1, :N, :]
                + b_ref[...][None, i * CB:(i + 1) * CB], 0.0)
             for i, p in enumerate(pr[...] for pr in p_refs)], axis=1)

    return pl.pallas_call(
        body,
        out_shape=jax.ShapeDtypeStruct((N, 128), jnp.float32),
    )(*p2, b2)


def _sc_pass_a(as1, ad1, src3, dst3):
    NA = as1.shape[0]

    @functools.partial(
        pl.kernel,
        out_type=(
            jax.ShapeDtypeStruct((NW, NCH, CHUNK), jnp.float32),
            jax.ShapeDtypeStruct((2, NP), jnp.float32),
        ),
        mesh=_mesh(),
        compiler_params=pltpu.CompilerParams(needs_layout_passes=False),
        scratch_types=[
            pltpu.VMEM((NA,), jnp.float32),
            pltpu.VMEM((NA,), jnp.float32),
            pltpu.VMEM((NCH, CHUNK), jnp.int32),
            pltpu.VMEM((NCH, CHUNK), jnp.int32),
            pltpu.VMEM((NCH, CHUNK), jnp.float32),
            pltpu.VMEM((NSLICE,), jnp.float32),
            pltpu.VMEM_SHARED((NP,), jnp.float32),
        ],
    )
    def kern(as_hbm, ad_hbm, src_hbm, dst_hbm, ex_hbm, d_hbm,
             as_v, ad_v, src_v, dst_v, ex_v, zero_v, den_sh):
        c = lax.axis_index("c")
        s = lax.axis_index("s")
        wid = s * 2 + c

        pltpu.sync_copy(as_hbm, as_v)
        pltpu.sync_copy(ad_hbm, ad_v)
        pltpu.sync_copy(src_hbm.at[wid], src_v)
        pltpu.sync_copy(dst_hbm.at[wid], dst_v)

        def zbody(t, _):
            zero_v[pl.ds(t * L, L)] = jnp.zeros((L,), jnp.float32)
            return _
        lax.fori_loop(0, NSLICE // L, zbody, None)
        pltpu.sync_copy(zero_v, den_sh.at[pl.ds(s * NSLICE, NSLICE)])
        plsc.subcore_barrier()

        def jbody(j, _):
            for k in range(CHUNK // L):
                isrc = src_v[j, pl.ds(k * L, L)]
                idst = dst_v[j, pl.ds(k * L, L)]
                vs = plsc.load_gather(as_v, [isrc])
                vd = plsc.load_gather(ad_v, [idst])
                e = vs + vd
                e = jnp.maximum(e, 0.2 * e)
                gid = (wid * EW + j * CHUNK + k * L
                       + lax.iota(jnp.int32, L))
                ex = jnp.where(gid < EE, jnp.exp(e), 0.0)
                ex_v[j, pl.ds(k * L, L)] = ex
            pltpu.sync_copy(ex_v.at[j], den_sh.at[dst_v.at[j]], add=True)
            return _
        lax.fori_loop(0, NCH, jbody, None)

        pltpu.sync_copy(ex_v, ex_hbm.at[wid])
        plsc.subcore_barrier()
        pltpu.sync_copy(den_sh.at[pl.ds(s * NSLICE, NSLICE)],
                        d_hbm.at[c, pl.ds(s * NSLICE, NSLICE)])

    return kern(as1, ad1, src3, dst3)


def _sc_pass_b(h_slabs, src3, dst3, ex, d):
    nslab = len(h_slabs)

    @functools.partial(
        pl.kernel,
        out_type=tuple(
            jax.ShapeDtypeStruct((2, NP, CB), jnp.float32)
            for _ in range(nslab)),
        mesh=_mesh(),
        compiler_params=pltpu.CompilerParams(
            needs_layout_passes=False, use_tc_tiling_on_sc=False),
        scratch_types=[
            pltpu.VMEM((NCH, CHUNK), jnp.int32),
            pltpu.VMEM((NCH, CHUNK), jnp.int32),
            pltpu.VMEM((NCH, CHUNK), jnp.float32),
            pltpu.VMEM((NCH, CHUNK), jnp.float32),
            pltpu.VMEM((NP,), jnp.float32),
            pltpu.VMEM((NP,), jnp.float32),
            pltpu.VMEM((CHUNK, CB), jnp.float32),
            pltpu.VMEM((CHUNK, CB), jnp.float32),
            pltpu.VMEM_SHARED((NP, CB), jnp.float32),
        ],
    )
    def kern(*refs):
        h_hbms = refs[:nslab]
        src_hbm, dst_hbm, ex_hbm, d_hbm = refs[nslab:nslab + 4]
        p_hbms = refs[nslab + 4:2 * nslab + 4]
        (src_v, dst_v, ex_v, w_v, den_v, tmp_v, rows_v, zb_v,
         out_sh) = refs[2 * nslab + 4:]
        c = lax.axis_index("c")
        s = lax.axis_index("s")
        wid = s * 2 + c

        pltpu.sync_copy(d_hbm.at[0], den_v)
        pltpu.sync_copy(d_hbm.at[1], tmp_v)
        pltpu.sync_copy(src_hbm.at[wid], src_v)
        pltpu.sync_copy(dst_hbm.at[wid], dst_v)
        pltpu.sync_copy(ex_hbm.at[wid], ex_v)

        def dbody(i, _):
            sl = pl.ds(i * L, L)
            den_v[sl] = den_v[sl] + tmp_v[sl]
            return _
        lax.fori_loop(0, NP // L, dbody, None)

        def wbody(j, _):
            for k in range(CHUNK // L):
                idst = dst_v[j, pl.ds(k * L, L)]
                den = plsc.load_gather(den_v, [idst])
                w_v[j, pl.ds(k * L, L)] = (
                    ex_v[j, pl.ds(k * L, L)] / (den + 1e-16))
            return _
        lax.fori_loop(0, NCH, wbody, None)

        def zbody(r, _):
            for k in range(CB // L):
                zb_v[r, pl.ds(k * L, L)] = jnp.zeros((L,), jnp.float32)
            return _
        lax.fori_loop(0, CHUNK, zbody, None)

        for si in range(nslab):
            for t in range(NSLICE // CHUNK):
                pltpu.sync_copy(
                    zb_v, out_sh.at[pl.ds(s * NSLICE + t * CHUNK, CHUNK), :])
            plsc.subcore_barrier()

            def jbody(j, _, h_hbm=h_hbms[si]):
                pltpu.sync_copy(h_hbm.at[src_v.at[j]], rows_v)

                def gbody(g, _):
                    wv16 = w_v[j, pl.ds(g * L, L)]
                    for rr in range(L):
                        r = g * L + rr
                        wv = jnp.full((L,), wv16[rr], jnp.float32)
                        for k in range(CB // L):
                            sl = pl.ds(k * L, L)
                            rows_v[r, sl] = rows_v[r, sl] * wv
                    return _
                lax.fori_loop(0, CHUNK // L, gbody, None)
                pltpu.sync_copy(rows_v, out_sh.at[dst_v.at[j]], add=True)
                return _
            lax.fori_loop(0, NCH, jbody, None)

            plsc.subcore_barrier()
            pltpu.sync_copy(out_sh.at[pl.ds(s * NSLICE, NSLICE), :],
                            p_hbms[si].at[c, pl.ds(s * NSLICE, NSLICE), :])

    return kern(*h_slabs, src3, dst3, ex, d)


def kernel(x, edge_index, W1, a_src1, a_dst1, b1, W2, a_src2, a_dst2, b2):
    loop = jnp.arange(N, dtype=jnp.int32)
    src = jnp.concatenate([edge_index[0].astype(jnp.int32), loop])
    dst = jnp.concatenate([edge_index[1].astype(jnp.int32), loop])
    src3 = jnp.pad(src, (0, EP - EE)).reshape(NW, NCH, CHUNK)
    dst3 = jnp.pad(dst, (0, EP - EE)).reshape(NW, NCH, CHUNK)

    *h1_slabs, as1, ad1 = _tc_layer1(x, W1, a_src1, a_dst1)
    ex1, d1 = _sc_pass_a(as1, ad1, src3, dst3)
    p1 = _sc_pass_b(h1_slabs, src3, dst3, ex1, d1)

    *h2_slabs, as2, ad2 = _tc_layer2(p1, b1, W2, a_src2, a_dst2)
    ex2, d2 = _sc_pass_a(as2, ad2, src3, dst3)
    p2 = _sc_pass_b(h2_slabs, src3, dst3, ex2, d2)

    return _tc_final(p2, b2)
